# Initial kernel scaffold; baseline (speedup 1.0000x reference)
import jax
import jax.numpy as jnp
from jax.experimental import pallas as pl


def kernel(x, edge_index, batch, demographic, W1, b1, W2, b2, W3, b3, fw1, fb1, fw2, fb2, fw3, fb3):
    raise NotImplementedError("write your pallas kernel here")


# speedup vs baseline: 9.6814x; 9.6814x over previous
import functools

import jax
import jax.numpy as jnp
from jax import lax
from jax.experimental import pallas as pl
from jax.experimental.pallas import tpu as pltpu
from jax.experimental.pallas import tpu_sc as plsc

N = 10000
E = 320000
D = 128
G = 100
GP = 104
NC = 2
NS = 16
NW = NC * NS
CH = 128
CPT = 79
EPT = CPT * CH
EPAD = EPT * NW
ZR = 640
NPAD = NS * ZR
RLAST = N - 15 * ZR
BK = 1000
NB = N // BK

_mesh = plsc.VectorSubcoreMesh(core_axis_name="c", subcore_axis_name="s")


@functools.partial(
    pl.kernel,
    mesh=_mesh,
    out_type=jax.ShapeDtypeStruct((NC * NPAD,), jnp.float32),
    scratch_types=[
        pltpu.VMEM((CPT, CH), jnp.int32),
        pltpu.VMEM((CH,), jnp.float32),
        pltpu.VMEM((ZR,), jnp.float32),
        pltpu.VMEM_SHARED((NPAD,), jnp.float32),
    ],
)
def _deg_call(dst_hbm, out_hbm, dst_v, ones_v, zbuf, hist):
    c = lax.axis_index("c")
    s = lax.axis_index("s")
    wid = c * NS + s

    for i in range(ZR // 16):
        zbuf[pl.ds(i * 16, 16)] = jnp.zeros((16,), jnp.float32)
    for i in range(CH // 16):
        ones_v[pl.ds(i * 16, 16)] = jnp.ones((16,), jnp.float32)

    pltpu.sync_copy(dst_hbm.at[wid], dst_v)
    pltpu.sync_copy(zbuf, hist.at[pl.ds(s * ZR, ZR)])
    plsc.subcore_barrier()

    def body(j, _):
        pltpu.sync_copy(ones_v, hist.at[dst_v.at[j]], add=True)
        return _

    lax.fori_loop(0, CPT, body, None)
    plsc.subcore_barrier()

    pltpu.sync_copy(hist.at[pl.ds(s * ZR, ZR)], zbuf)
    pltpu.sync_copy(zbuf, out_hbm.at[pl.ds(c * NPAD + s * ZR, ZR)])


@functools.partial(
    pl.kernel,
    mesh=_mesh,
    out_type=jax.ShapeDtypeStruct((NC * N, D), jnp.float32),
    scratch_types=[
        pltpu.VMEM((CPT, CH), jnp.int32),
        pltpu.VMEM((CPT, CH), jnp.int32),
        pltpu.VMEM((CH, D), jnp.float32),
        pltpu.VMEM((8, D), jnp.float32),
        pltpu.VMEM_SHARED((NPAD, D), jnp.float32),
        pltpu.SemaphoreType.DMA,
    ],
)
def _msg_call(hws_hbm, src_hbm, dst_hbm, out_hbm,
              src_v, dst_v, rows_v, zbuf, acc, sem):
    c = lax.axis_index("c")
    s = lax.axis_index("s")
    wid = c * NS + s

    for i in range(8):
        for j in range(D // 16):
            zbuf[i, pl.ds(j * 16, 16)] = jnp.zeros((16,), jnp.float32)

    pltpu.sync_copy(src_hbm.at[wid], src_v)
    pltpu.sync_copy(dst_hbm.at[wid], dst_v)

    def zero_body(k, _):
        pltpu.sync_copy(zbuf, acc.at[pl.ds(s * ZR + k * 8, 8)])
        return _

    lax.fori_loop(0, ZR // 8, zero_body, None)
    plsc.subcore_barrier()

    def body(j, _):
        pltpu.async_copy(hws_hbm.at[src_v.at[j]], rows_v, sem).wait()
        pltpu.sync_copy(rows_v, acc.at[dst_v.at[j]], add=True)
        return _

    lax.fori_loop(0, CPT, body, None)
    plsc.subcore_barrier()

    @pl.when(s < NS - 1)
    def _wb():
        pltpu.sync_copy(acc.at[pl.ds(s * ZR, ZR)],
                        out_hbm.at[pl.ds(c * N + s * ZR, ZR)])

    @pl.when(s == NS - 1)
    def _wb_last():
        pltpu.sync_copy(acc.at[pl.ds(s * ZR, RLAST)],
                        out_hbm.at[pl.ds(c * N + s * ZR, RLAST)])


def _dense1_body(d0_ref, d1_ref, x_ref, w_ref, hw_ref, hws_ref):
    deg = d0_ref[...] + d1_ref[...] + 1.0
    dis = lax.rsqrt(deg)
    hw = jnp.dot(x_ref[...], w_ref[...], preferred_element_type=jnp.float32)
    hw_ref[...] = hw
    hws_ref[...] = hw * dis


def _dense1(d0, d1, x, w):
    return pl.pallas_call(
        _dense1_body,
        grid=(NB,),
        in_specs=[
            pl.BlockSpec((BK, 1), lambda i: (i, 0)),
            pl.BlockSpec((BK, 1), lambda i: (i, 0)),
            pl.BlockSpec((BK, D), lambda i: (i, 0)),
            pl.BlockSpec((D, D), lambda i: (0, 0)),
        ],
        out_specs=[
            pl.BlockSpec((BK, D), lambda i: (i, 0)),
            pl.BlockSpec((BK, D), lambda i: (i, 0)),
        ],
        out_shape=[
            jax.ShapeDtypeStruct((N, D), jnp.float32),
            jax.ShapeDtypeStruct((N, D), jnp.float32),
        ],
    )(d0, d1, x, w)


def _mid_body(d0_ref, d1_ref, p0_ref, p1_ref, hwp_ref, b_ref, w_ref,
              hw_ref, hws_ref):
    deg = d0_ref[...] + d1_ref[...] + 1.0
    dis = lax.rsqrt(deg)
    agg = dis * (p0_ref[...] + p1_ref[...]) + (dis * dis) * hwp_ref[...]
    h = jnp.maximum(agg + b_ref[...], 0.0)
    hw = jnp.dot(h, w_ref[...], preferred_element_type=jnp.float32)
    hw_ref[...] = hw
    hws_ref[...] = hw * dis


def _mid(d0, d1, p0, p1, hwp, b, w):
    return pl.pallas_call(
        _mid_body,
        grid=(NB,),
        in_specs=[
            pl.BlockSpec((BK, 1), lambda i: (i, 0)),
            pl.BlockSpec((BK, 1), lambda i: (i, 0)),
            pl.BlockSpec((BK, D), lambda i: (i, 0)),
            pl.BlockSpec((BK, D), lambda i: (i, 0)),
            pl.BlockSpec((BK, D), lambda i: (i, 0)),
            pl.BlockSpec((1, D), lambda i: (0, 0)),
            pl.BlockSpec((D, D), lambda i: (0, 0)),
        ],
        out_specs=[
            pl.BlockSpec((BK, D), lambda i: (i, 0)),
            pl.BlockSpec((BK, D), lambda i: (i, 0)),
        ],
        out_shape=[
            jax.ShapeDtypeStruct((N, D), jnp.float32),
            jax.ShapeDtypeStruct((N, D), jnp.float32),
        ],
    )(d0, d1, p0, p1, hwp, b, w)


def _head_body(d0_ref, d1_ref, p0_ref, p1_ref, hwp_ref, b_ref, batch_ref,
               demo_ref, fw1a_ref, fw1b_ref, fb1_ref, fw2_ref, fb2_ref,
               fw3_ref, fb3_ref, out_ref, sum_acc, cnt_acc):
    i = pl.program_id(0)

    @pl.when(i == 0)
    def _init():
        sum_acc[...] = jnp.zeros_like(sum_acc)
        cnt_acc[...] = jnp.zeros_like(cnt_acc)

    deg = d0_ref[...] + d1_ref[...] + 1.0
    dis = lax.rsqrt(deg)
    agg = dis * (p0_ref[...] + p1_ref[...]) + (dis * dis) * hwp_ref[...]
    h = jnp.maximum(agg + b_ref[...], 0.0)

    gid = lax.broadcasted_iota(jnp.int32, (GP, 1), 0)
    mask = (gid == batch_ref[...].reshape(1, BK)).astype(jnp.float32)
    sum_acc[...] += jnp.dot(mask, h, preferred_element_type=jnp.float32)
    cnt_acc[...] += jnp.sum(mask, axis=1, keepdims=True)

    @pl.when(i == NB - 1)
    def _fin():
        pooled = sum_acc[...] / jnp.maximum(cnt_acc[...], 1.0)
        z = jnp.dot(pooled, fw1a_ref[...], preferred_element_type=jnp.float32)
        z += jnp.dot(demo_ref[...], fw1b_ref[...],
                     preferred_element_type=jnp.float32)
        z = jnp.maximum(z + fb1_ref[...], 0.0)
        z = jnp.maximum(
            jnp.dot(z, fw2_ref[...], preferred_element_type=jnp.float32)
            + fb2_ref[...], 0.0)
        z = jnp.dot(z, fw3_ref[...],
                    preferred_element_type=jnp.float32) + fb3_ref[...]
        out_ref[...] = z[:G]


def _head(d0, d1, p0, p1, hwp, b, batch2d, demo,
          fw1a, fw1b, fb1, fw2, fb2, fw3, fb3):
    return pl.pallas_call(
        _head_body,
        grid=(NB,),
        in_specs=[
            pl.BlockSpec((BK, 1), lambda i: (i, 0)),
            pl.BlockSpec((BK, 1), lambda i: (i, 0)),
            pl.BlockSpec((BK, D), lambda i: (i, 0)),
            pl.BlockSpec((BK, D), lambda i: (i, 0)),
            pl.BlockSpec((BK, D), lambda i: (i, 0)),
            pl.BlockSpec((1, D), lambda i: (0, 0)),
            pl.BlockSpec((1, 1, BK), lambda i: (i, 0, 0)),
            pl.BlockSpec((GP, 8), lambda i: (0, 0)),
            pl.BlockSpec((D, 64), lambda i: (0, 0)),
            pl.BlockSpec((8, 64), lambda i: (0, 0)),
            pl.BlockSpec((1, 64), lambda i: (0, 0)),
            pl.BlockSpec((64, 32), lambda i: (0, 0)),
            pl.BlockSpec((1, 32), lambda i: (0, 0)),
            pl.BlockSpec((32, 2), lambda i: (0, 0)),
            pl.BlockSpec((1, 2), lambda i: (0, 0)),
        ],
        out_specs=pl.BlockSpec((G, 2), lambda i: (0, 0)),
        out_shape=jax.ShapeDtypeStruct((G, 2), jnp.float32),
        scratch_shapes=[
            pltpu.VMEM((GP, D), jnp.float32),
            pltpu.VMEM((GP, 1), jnp.float32),
        ],
    )(d0, d1, p0, p1, hwp, b, batch2d, demo,
      fw1a, fw1b, fb1, fw2, fb2, fw3, fb3)


def kernel(x, edge_index, batch, demographic,
           W1, b1, W2, b2, W3, b3, fw1, fb1, fw2, fb2, fw3, fb3):
    src = edge_index[0].astype(jnp.int32)
    dst = edge_index[1].astype(jnp.int32)
    pad = EPAD - E
    src_p = jnp.concatenate(
        [src, jnp.zeros((pad,), jnp.int32)]).reshape(NW, CPT, CH)
    dst_p = jnp.concatenate(
        [dst, jnp.full((pad,), N, jnp.int32)]).reshape(NW, CPT, CH)

    degp = _deg_call(dst_p)
    d0 = degp[:N].reshape(N, 1)
    d1 = degp[NPAD:NPAD + N].reshape(N, 1)

    hw1, hws1 = _dense1(d0, d1, x, W1)
    s1 = _msg_call(hws1, src_p, dst_p)
    hw2, hws2 = _mid(d0, d1, s1[:N], s1[N:], hw1, b1.reshape(1, D), W2)
    s2 = _msg_call(hws2, src_p, dst_p)
    hw3, hws3 = _mid(d0, d1, s2[:N], s2[N:], hw2, b2.reshape(1, D), W3)
    s3 = _msg_call(hws3, src_p, dst_p)

    batch2d = batch.reshape(NB, 1, BK).astype(jnp.int32)
    demo = jnp.zeros((GP, 8), jnp.float32).at[:G].set(demographic)
    return _head(d0, d1, s3[:N], s3[N:], hw3, b3.reshape(1, D), batch2d,
                 demo, fw1[:D], fw1[D:], fb1.reshape(1, 64),
                 fw2, fb2.reshape(1, 32), fw3, fb3.reshape(1, 2))


# speedup vs baseline: 26.6734x; 2.7551x over previous
import functools

import jax
import jax.numpy as jnp
from jax import lax
from jax.experimental import pallas as pl
from jax.experimental.pallas import tpu as pltpu
from jax.experimental.pallas import tpu_sc as plsc

N = 10000
E = 320000
D = 128
G = 100
GP = 104
NC = 2
NS = 16
NW = NC * NS
CH = 128
CPT = 80
EPT = CPT * CH
EPAD = EPT * NW
ZRM = 632
NPADM = NS * ZRM
RLASTM = N - 15 * ZRM
ZRD = 640
NPADD = NS * ZRD
BK = 1000
NB = N // BK

_mesh = plsc.VectorSubcoreMesh(core_axis_name="c", subcore_axis_name="s")


@functools.partial(
    pl.kernel,
    mesh=_mesh,
    out_type=jax.ShapeDtypeStruct((NC * NPADD,), jnp.float32),
    scratch_types=[
        pltpu.VMEM((CPT, CH), jnp.int32),
        pltpu.VMEM((CH,), jnp.float32),
        pltpu.VMEM((ZRD,), jnp.float32),
        pltpu.VMEM_SHARED((NPADD,), jnp.float32),
    ],
)
def _deg_call(dst_hbm, out_hbm, dst_v, ones_v, zbuf, hist):
    c = lax.axis_index("c")
    s = lax.axis_index("s")
    wid = c * NS + s

    for i in range(ZRD // 16):
        zbuf[pl.ds(i * 16, 16)] = jnp.zeros((16,), jnp.float32)
    for i in range(CH // 16):
        ones_v[pl.ds(i * 16, 16)] = jnp.ones((16,), jnp.float32)

    pltpu.sync_copy(dst_hbm.at[wid], dst_v)
    pltpu.sync_copy(zbuf, hist.at[pl.ds(s * ZRD, ZRD)])
    plsc.subcore_barrier()

    def body(j, _):
        pltpu.sync_copy(ones_v, hist.at[dst_v.at[j]], add=True)
        return _

    lax.fori_loop(0, CPT, body, None)
    plsc.subcore_barrier()

    pltpu.sync_copy(hist.at[pl.ds(s * ZRD, ZRD)], zbuf)
    pltpu.sync_copy(zbuf, out_hbm.at[pl.ds(c * NPADD + s * ZRD, ZRD)])


@functools.partial(
    pl.kernel,
    mesh=_mesh,
    out_type=jax.ShapeDtypeStruct((NC * N, D), jnp.float32),
    scratch_types=[
        pltpu.VMEM((CPT, CH), jnp.int32),
        pltpu.VMEM((CH,), jnp.int32),
        pltpu.VMEM((CH,), jnp.int32),
        pltpu.VMEM((CH, D), jnp.float32),
        pltpu.VMEM((CH, D), jnp.float32),
        pltpu.VMEM((8, D), jnp.float32),
        pltpu.VMEM_SHARED((NPADM, D), jnp.float32),
        pltpu.SemaphoreType.DMA,
        pltpu.SemaphoreType.DMA,
        pltpu.SemaphoreType.DMA,
        pltpu.SemaphoreType.DMA,
    ],
)
def _msg_call(hws_hbm, src_hbm, dst_hbm, out_hbm,
              src_v, didx0, didx1, rows0, rows1, zbuf, acc,
              sem0, sem1, dsem0, dsem1):
    c = lax.axis_index("c")
    s = lax.axis_index("s")
    wid = c * NS + s

    for i in range(8):
        for j in range(D // 16):
            zbuf[i, pl.ds(j * 16, 16)] = jnp.zeros((16,), jnp.float32)

    pltpu.sync_copy(src_hbm.at[wid], src_v)

    def zero_body(k, _):
        pltpu.sync_copy(zbuf, acc.at[pl.ds(s * ZRM + k * 8, 8)])
        return _

    lax.fori_loop(0, ZRM // 8, zero_body, None)
    plsc.subcore_barrier()

    pltpu.async_copy(dst_hbm.at[wid, 0], didx0, dsem0)
    pltpu.async_copy(hws_hbm.at[src_v.at[0]], rows0, sem0)

    def body(k, _):
        j0 = 2 * k
        pltpu.async_copy(dst_hbm.at[wid, j0 + 1], didx1, dsem1)
        pltpu.async_copy(hws_hbm.at[src_v.at[j0 + 1]], rows1, sem1)
        pltpu.make_async_copy(dst_hbm.at[wid, j0], didx0, dsem0).wait()
        pltpu.make_async_copy(hws_hbm.at[src_v.at[j0]], rows0, sem0).wait()
        pltpu.sync_copy(rows0, acc.at[didx0], add=True)

        @pl.when(k < CPT // 2 - 1)
        def _():
            pltpu.async_copy(dst_hbm.at[wid, j0 + 2], didx0, dsem0)
            pltpu.async_copy(hws_hbm.at[src_v.at[j0 + 2]], rows0, sem0)

        pltpu.make_async_copy(dst_hbm.at[wid, j0 + 1], didx1, dsem1).wait()
        pltpu.make_async_copy(hws_hbm.at[src_v.at[j0 + 1]], rows1, sem1).wait()
        pltpu.sync_copy(rows1, acc.at[didx1], add=True)
        return _

    lax.fori_loop(0, CPT // 2, body, None)
    plsc.subcore_barrier()

    @pl.when(s < NS - 1)
    def _wb():
        pltpu.sync_copy(acc.at[pl.ds(s * ZRM, ZRM)],
                        out_hbm.at[pl.ds(c * N + s * ZRM, ZRM)])

    @pl.when(s == NS - 1)
    def _wb_last():
        pltpu.sync_copy(acc.at[pl.ds(s * ZRM, RLASTM)],
                        out_hbm.at[pl.ds(c * N + s * ZRM, RLASTM)])


def _dense1_body(d0_ref, d1_ref, x_ref, w_ref, hw_ref, hws_ref):
    deg = d0_ref[...] + d1_ref[...] + 1.0
    dis = lax.rsqrt(deg)
    hw = jnp.dot(x_ref[...], w_ref[...], preferred_element_type=jnp.float32)
    hw_ref[...] = hw
    hws_ref[...] = hw * dis


def _dense1(d0, d1, x, w):
    return pl.pallas_call(
        _dense1_body,
        grid=(NB,),
        in_specs=[
            pl.BlockSpec((BK, 1), lambda i: (i, 0)),
            pl.BlockSpec((BK, 1), lambda i: (i, 0)),
            pl.BlockSpec((BK, D), lambda i: (i, 0)),
            pl.BlockSpec((D, D), lambda i: (0, 0)),
        ],
        out_specs=[
            pl.BlockSpec((BK, D), lambda i: (i, 0)),
            pl.BlockSpec((BK, D), lambda i: (i, 0)),
        ],
        out_shape=[
            jax.ShapeDtypeStruct((N, D), jnp.float32),
            jax.ShapeDtypeStruct((N, D), jnp.float32),
        ],
    )(d0, d1, x, w)


def _mid_body(d0_ref, d1_ref, p0_ref, p1_ref, hwp_ref, b_ref, w_ref,
              hw_ref, hws_ref):
    deg = d0_ref[...] + d1_ref[...] + 1.0
    dis = lax.rsqrt(deg)
    agg = dis * (p0_ref[...] + p1_ref[...]) + (dis * dis) * hwp_ref[...]
    h = jnp.maximum(agg + b_ref[...], 0.0)
    hw = jnp.dot(h, w_ref[...], preferred_element_type=jnp.float32)
    hw_ref[...] = hw
    hws_ref[...] = hw * dis


def _mid(d0, d1, p0, p1, hwp, b, w):
    return pl.pallas_call(
        _mid_body,
        grid=(NB,),
        in_specs=[
            pl.BlockSpec((BK, 1), lambda i: (i, 0)),
            pl.BlockSpec((BK, 1), lambda i: (i, 0)),
            pl.BlockSpec((BK, D), lambda i: (i, 0)),
            pl.BlockSpec((BK, D), lambda i: (i, 0)),
            pl.BlockSpec((BK, D), lambda i: (i, 0)),
            pl.BlockSpec((1, D), lambda i: (0, 0)),
            pl.BlockSpec((D, D), lambda i: (0, 0)),
        ],
        out_specs=[
            pl.BlockSpec((BK, D), lambda i: (i, 0)),
            pl.BlockSpec((BK, D), lambda i: (i, 0)),
        ],
        out_shape=[
            jax.ShapeDtypeStruct((N, D), jnp.float32),
            jax.ShapeDtypeStruct((N, D), jnp.float32),
        ],
    )(d0, d1, p0, p1, hwp, b, w)


def _head_body(d0_ref, d1_ref, p0_ref, p1_ref, hwp_ref, b_ref, batch_ref,
               demo_ref, fw1a_ref, fw1b_ref, fb1_ref, fw2_ref, fb2_ref,
               fw3_ref, fb3_ref, out_ref, sum_acc, cnt_acc):
    i = pl.program_id(0)

    @pl.when(i == 0)
    def _init():
        sum_acc[...] = jnp.zeros_like(sum_acc)
        cnt_acc[...] = jnp.zeros_like(cnt_acc)

    deg = d0_ref[...] + d1_ref[...] + 1.0
    dis = lax.rsqrt(deg)
    agg = dis * (p0_ref[...] + p1_ref[...]) + (dis * dis) * hwp_ref[...]
    h = jnp.maximum(agg + b_ref[...], 0.0)

    gid = lax.broadcasted_iota(jnp.int32, (GP, 1), 0)
    mask = (gid == batch_ref[...].reshape(1, BK)).astype(jnp.float32)
    sum_acc[...] += jnp.dot(mask, h, preferred_element_type=jnp.float32)
    cnt_acc[...] += jnp.sum(mask, axis=1, keepdims=True)

    @pl.when(i == NB - 1)
    def _fin():
        pooled = sum_acc[...] / jnp.maximum(cnt_acc[...], 1.0)
        z = jnp.dot(pooled, fw1a_ref[...], preferred_element_type=jnp.float32)
        z += jnp.dot(demo_ref[...], fw1b_ref[...],
                     preferred_element_type=jnp.float32)
        z = jnp.maximum(z + fb1_ref[...], 0.0)
        z = jnp.maximum(
            jnp.dot(z, fw2_ref[...], preferred_element_type=jnp.float32)
            + fb2_ref[...], 0.0)
        z = jnp.dot(z, fw3_ref[...],
                    preferred_element_type=jnp.float32) + fb3_ref[...]
        out_ref[...] = z[:G]


def _head(d0, d1, p0, p1, hwp, b, batch2d, demo,
          fw1a, fw1b, fb1, fw2, fb2, fw3, fb3):
    return pl.pallas_call(
        _head_body,
        grid=(NB,),
        in_specs=[
            pl.BlockSpec((BK, 1), lambda i: (i, 0)),
            pl.BlockSpec((BK, 1), lambda i: (i, 0)),
            pl.BlockSpec((BK, D), lambda i: (i, 0)),
            pl.BlockSpec((BK, D), lambda i: (i, 0)),
            pl.BlockSpec((BK, D), lambda i: (i, 0)),
            pl.BlockSpec((1, D), lambda i: (0, 0)),
            pl.BlockSpec((1, 1, BK), lambda i: (i, 0, 0)),
            pl.BlockSpec((GP, 8), lambda i: (0, 0)),
            pl.BlockSpec((D, 64), lambda i: (0, 0)),
            pl.BlockSpec((8, 64), lambda i: (0, 0)),
            pl.BlockSpec((1, 64), lambda i: (0, 0)),
            pl.BlockSpec((64, 32), lambda i: (0, 0)),
            pl.BlockSpec((1, 32), lambda i: (0, 0)),
            pl.BlockSpec((32, 2), lambda i: (0, 0)),
            pl.BlockSpec((1, 2), lambda i: (0, 0)),
        ],
        out_specs=pl.BlockSpec((G, 2), lambda i: (0, 0)),
        out_shape=jax.ShapeDtypeStruct((G, 2), jnp.float32),
        scratch_shapes=[
            pltpu.VMEM((GP, D), jnp.float32),
            pltpu.VMEM((GP, 1), jnp.float32),
        ],
    )(d0, d1, p0, p1, hwp, b, batch2d, demo,
      fw1a, fw1b, fb1, fw2, fb2, fw3, fb3)


def kernel(x, edge_index, batch, demographic,
           W1, b1, W2, b2, W3, b3, fw1, fb1, fw2, fb2, fw3, fb3):
    src = edge_index[0].astype(jnp.int32)
    dst = edge_index[1].astype(jnp.int32)
    pad = EPAD - E
    pad_i = jnp.arange(pad, dtype=jnp.int32)
    src_p = jnp.concatenate(
        [src, pad_i % N]).reshape(NW, CPT, CH)
    dst_p = jnp.concatenate(
        [dst, N + pad_i % (NPADM - N)]).reshape(NW, CPT, CH)

    degp = _deg_call(dst_p)
    d0 = degp[:N].reshape(N, 1)
    d1 = degp[NPADD:NPADD + N].reshape(N, 1)

    hw1, hws1 = _dense1(d0, d1, x, W1)
    s1 = _msg_call(hws1, src_p, dst_p)
    hw2, hws2 = _mid(d0, d1, s1[:N], s1[N:], hw1, b1.reshape(1, D), W2)
    s2 = _msg_call(hws2, src_p, dst_p)
    hw3, hws3 = _mid(d0, d1, s2[:N], s2[N:], hw2, b2.reshape(1, D), W3)
    s3 = _msg_call(hws3, src_p, dst_p)

    batch2d = batch.reshape(NB, 1, BK).astype(jnp.int32)
    demo = jnp.zeros((GP, 8), jnp.float32).at[:G].set(demographic)
    return _head(d0, d1, s3[:N], s3[N:], hw3, b3.reshape(1, D), batch2d,
                 demo, fw1[:D], fw1[D:], fb1.reshape(1, 64),
                 fw2, fb2.reshape(1, 32), fw3, fb3.reshape(1, 2))


# speedup vs baseline: 27.8923x; 1.0457x over previous
import functools

import jax
import jax.numpy as jnp
from jax import lax
from jax.experimental import pallas as pl
from jax.experimental.pallas import tpu as pltpu
from jax.experimental.pallas import tpu_sc as plsc

N = 10000
E = 320000
D = 128
G = 100
GP = 104
NC = 2
NS = 16
NW = NC * NS
CH = 128
CPT = 80
EPT = CPT * CH
EPAD = EPT * NW
ZRM = 632
NPADM = NS * ZRM
RLASTM = N - 15 * ZRM
ZRD = 640
NPADD = NS * ZRD
BK = 1000
NB = N // BK

_mesh = plsc.VectorSubcoreMesh(core_axis_name="c", subcore_axis_name="s")


@functools.partial(
    pl.kernel,
    mesh=_mesh,
    out_type=jax.ShapeDtypeStruct((NC * NPADD,), jnp.float32),
    scratch_types=[
        pltpu.VMEM((CPT, CH), jnp.int32),
        pltpu.VMEM((CH,), jnp.float32),
        pltpu.VMEM((ZRD,), jnp.float32),
        pltpu.VMEM_SHARED((NPADD,), jnp.float32),
    ],
)
def _deg_call(dst_hbm, out_hbm, dst_v, ones_v, zbuf, hist):
    c = lax.axis_index("c")
    s = lax.axis_index("s")
    wid = c * NS + s

    for i in range(ZRD // 16):
        zbuf[pl.ds(i * 16, 16)] = jnp.zeros((16,), jnp.float32)
    for i in range(CH // 16):
        ones_v[pl.ds(i * 16, 16)] = jnp.ones((16,), jnp.float32)

    pltpu.sync_copy(dst_hbm.at[wid], dst_v)
    pltpu.sync_copy(zbuf, hist.at[pl.ds(s * ZRD, ZRD)])
    plsc.subcore_barrier()

    def body(j, _):
        pltpu.sync_copy(ones_v, hist.at[dst_v.at[j]], add=True)
        return _

    lax.fori_loop(0, CPT, body, None)
    plsc.subcore_barrier()

    pltpu.sync_copy(hist.at[pl.ds(s * ZRD, ZRD)], zbuf)
    pltpu.sync_copy(zbuf, out_hbm.at[pl.ds(c * NPADD + s * ZRD, ZRD)])


@functools.partial(
    pl.kernel,
    mesh=_mesh,
    out_type=jax.ShapeDtypeStruct((NC * N, D), jnp.float32),
    scratch_types=[
        pltpu.VMEM((CPT, CH), jnp.int32),
        pltpu.VMEM((CH,), jnp.int32),
        pltpu.VMEM((CH,), jnp.int32),
        pltpu.VMEM((CH, D), jnp.float32),
        pltpu.VMEM((CH, D), jnp.float32),
        pltpu.VMEM((8, D), jnp.float32),
        pltpu.VMEM_SHARED((NPADM, D), jnp.float32),
        pltpu.SemaphoreType.DMA,
        pltpu.SemaphoreType.DMA,
        pltpu.SemaphoreType.DMA,
        pltpu.SemaphoreType.DMA,
    ],
)
def _msg_call(hws_hbm, src_hbm, dst_hbm, out_hbm,
              src_v, didx0, didx1, rows0, rows1, zbuf, acc,
              sem0, sem1, dsem0, dsem1):
    c = lax.axis_index("c")
    s = lax.axis_index("s")
    wid = c * NS + s

    for i in range(8):
        for j in range(D // 16):
            zbuf[i, pl.ds(j * 16, 16)] = jnp.zeros((16,), jnp.float32)

    pltpu.sync_copy(src_hbm.at[wid], src_v)

    def zero_body(k, _):
        pltpu.sync_copy(zbuf, acc.at[pl.ds(s * ZRM + k * 8, 8)])
        return _

    lax.fori_loop(0, ZRM // 8, zero_body, None)
    plsc.subcore_barrier()

    pltpu.async_copy(dst_hbm.at[wid, 0], didx0, dsem0)
    pltpu.async_copy(hws_hbm.at[src_v.at[0]], rows0, sem0)

    def body(k, _):
        j0 = 2 * k
        pltpu.async_copy(dst_hbm.at[wid, j0 + 1], didx1, dsem1)
        pltpu.async_copy(hws_hbm.at[src_v.at[j0 + 1]], rows1, sem1)
        pltpu.make_async_copy(dst_hbm.at[wid, j0], didx0, dsem0).wait()
        pltpu.make_async_copy(hws_hbm.at[src_v.at[j0]], rows0, sem0).wait()
        pltpu.sync_copy(rows0, acc.at[didx0], add=True)

        @pl.when(k < CPT // 2 - 1)
        def _():
            pltpu.async_copy(dst_hbm.at[wid, j0 + 2], didx0, dsem0)
            pltpu.async_copy(hws_hbm.at[src_v.at[j0 + 2]], rows0, sem0)

        pltpu.make_async_copy(dst_hbm.at[wid, j0 + 1], didx1, dsem1).wait()
        pltpu.make_async_copy(hws_hbm.at[src_v.at[j0 + 1]], rows1, sem1).wait()
        pltpu.sync_copy(rows1, acc.at[didx1], add=True)
        return _

    lax.fori_loop(0, CPT // 2, body, None)
    plsc.subcore_barrier()

    @pl.when(s < NS - 1)
    def _wb():
        pltpu.sync_copy(acc.at[pl.ds(s * ZRM, ZRM)],
                        out_hbm.at[pl.ds(c * N + s * ZRM, ZRM)])

    @pl.when(s == NS - 1)
    def _wb_last():
        pltpu.sync_copy(acc.at[pl.ds(s * ZRM, RLASTM)],
                        out_hbm.at[pl.ds(c * N + s * ZRM, RLASTM)])


def _dense1_body(d0_ref, d1_ref, x_ref, w_ref, hw_ref, hws_ref):
    deg = d0_ref[...] + d1_ref[...] + 1.0
    dis = lax.rsqrt(deg)
    hw = jnp.dot(x_ref[...], w_ref[...], preferred_element_type=jnp.float32)
    hw_ref[...] = hw
    hws_ref[...] = hw * dis


def _dense1(d0, d1, x, w):
    return pl.pallas_call(
        _dense1_body,
        grid=(NB,),
        in_specs=[
            pl.BlockSpec((BK, 1), lambda i: (i, 0)),
            pl.BlockSpec((BK, 1), lambda i: (i, 0)),
            pl.BlockSpec((BK, D), lambda i: (i, 0)),
            pl.BlockSpec((D, D), lambda i: (0, 0)),
        ],
        out_specs=[
            pl.BlockSpec((BK, D), lambda i: (i, 0)),
            pl.BlockSpec((BK, D), lambda i: (i, 0)),
        ],
        out_shape=[
            jax.ShapeDtypeStruct((N, D), jnp.float32),
            jax.ShapeDtypeStruct((N, D), jnp.float32),
        ],
    )(d0, d1, x, w)


def _mid_body(d0_ref, d1_ref, p0_ref, p1_ref, hwp_ref, b_ref, w_ref,
              hw_ref, hws_ref):
    deg = d0_ref[...] + d1_ref[...] + 1.0
    dis = lax.rsqrt(deg)
    agg = dis * (p0_ref[...] + p1_ref[...]) + (dis * dis) * hwp_ref[...]
    h = jnp.maximum(agg + b_ref[...], 0.0)
    hw = jnp.dot(h, w_ref[...], preferred_element_type=jnp.float32)
    hw_ref[...] = hw
    hws_ref[...] = hw * dis


def _mid(d0, d1, p0, p1, hwp, b, w):
    return pl.pallas_call(
        _mid_body,
        grid=(NB,),
        in_specs=[
            pl.BlockSpec((BK, 1), lambda i: (i, 0)),
            pl.BlockSpec((BK, 1), lambda i: (i, 0)),
            pl.BlockSpec((BK, D), lambda i: (i, 0)),
            pl.BlockSpec((BK, D), lambda i: (i + NB, 0)),
            pl.BlockSpec((BK, D), lambda i: (i, 0)),
            pl.BlockSpec((1, D), lambda i: (0, 0)),
            pl.BlockSpec((D, D), lambda i: (0, 0)),
        ],
        out_specs=[
            pl.BlockSpec((BK, D), lambda i: (i, 0)),
            pl.BlockSpec((BK, D), lambda i: (i, 0)),
        ],
        out_shape=[
            jax.ShapeDtypeStruct((N, D), jnp.float32),
            jax.ShapeDtypeStruct((N, D), jnp.float32),
        ],
    )(d0, d1, p0, p1, hwp, b, w)


def _head_body(d0_ref, d1_ref, p0_ref, p1_ref, hwp_ref, b_ref, batch_ref,
               demo_ref, fw1a_ref, fw1b_ref, fb1_ref, fw2_ref, fb2_ref,
               fw3_ref, fb3_ref, out_ref, sum_acc, cnt_acc):
    i = pl.program_id(0)

    @pl.when(i == 0)
    def _init():
        sum_acc[...] = jnp.zeros_like(sum_acc)
        cnt_acc[...] = jnp.zeros_like(cnt_acc)

    deg = d0_ref[...] + d1_ref[...] + 1.0
    dis = lax.rsqrt(deg)
    agg = dis * (p0_ref[...] + p1_ref[...]) + (dis * dis) * hwp_ref[...]
    h = jnp.maximum(agg + b_ref[...], 0.0)

    gid = lax.broadcasted_iota(jnp.int32, (GP, 1), 0)
    mask = (gid == batch_ref[...].reshape(1, BK)).astype(jnp.float32)
    sum_acc[...] += jnp.dot(mask, h, preferred_element_type=jnp.float32)
    cnt_acc[...] += jnp.sum(mask, axis=1, keepdims=True)

    @pl.when(i == NB - 1)
    def _fin():
        pooled = sum_acc[...] / jnp.maximum(cnt_acc[...], 1.0)
        z = jnp.dot(pooled, fw1a_ref[...], preferred_element_type=jnp.float32)
        z += jnp.dot(demo_ref[...], fw1b_ref[...],
                     preferred_element_type=jnp.float32)
        z = jnp.maximum(z + fb1_ref[...], 0.0)
        z = jnp.maximum(
            jnp.dot(z, fw2_ref[...], preferred_element_type=jnp.float32)
            + fb2_ref[...], 0.0)
        z = jnp.dot(z, fw3_ref[...],
                    preferred_element_type=jnp.float32) + fb3_ref[...]
        out_ref[...] = z[:G]


def _head(d0, d1, p0, p1, hwp, b, batch2d, demo,
          fw1a, fw1b, fb1, fw2, fb2, fw3, fb3):
    return pl.pallas_call(
        _head_body,
        grid=(NB,),
        in_specs=[
            pl.BlockSpec((BK, 1), lambda i: (i, 0)),
            pl.BlockSpec((BK, 1), lambda i: (i, 0)),
            pl.BlockSpec((BK, D), lambda i: (i, 0)),
            pl.BlockSpec((BK, D), lambda i: (i + NB, 0)),
            pl.BlockSpec((BK, D), lambda i: (i, 0)),
            pl.BlockSpec((1, D), lambda i: (0, 0)),
            pl.BlockSpec((1, 1, BK), lambda i: (i, 0, 0)),
            pl.BlockSpec((GP, 8), lambda i: (0, 0)),
            pl.BlockSpec((D, 64), lambda i: (0, 0)),
            pl.BlockSpec((8, 64), lambda i: (0, 0)),
            pl.BlockSpec((1, 64), lambda i: (0, 0)),
            pl.BlockSpec((64, 32), lambda i: (0, 0)),
            pl.BlockSpec((1, 32), lambda i: (0, 0)),
            pl.BlockSpec((32, 2), lambda i: (0, 0)),
            pl.BlockSpec((1, 2), lambda i: (0, 0)),
        ],
        out_specs=pl.BlockSpec((G, 2), lambda i: (0, 0)),
        out_shape=jax.ShapeDtypeStruct((G, 2), jnp.float32),
        scratch_shapes=[
            pltpu.VMEM((GP, D), jnp.float32),
            pltpu.VMEM((GP, 1), jnp.float32),
        ],
    )(d0, d1, p0, p1, hwp, b, batch2d, demo,
      fw1a, fw1b, fb1, fw2, fb2, fw3, fb3)


def kernel(x, edge_index, batch, demographic,
           W1, b1, W2, b2, W3, b3, fw1, fb1, fw2, fb2, fw3, fb3):
    src = edge_index[0].astype(jnp.int32)
    dst = edge_index[1].astype(jnp.int32)
    pad = EPAD - E
    pad_i = jnp.arange(pad, dtype=jnp.int32)
    src_p = jnp.concatenate(
        [src, pad_i % N]).reshape(NW, CPT, CH)
    dst_p = jnp.concatenate(
        [dst, N + pad_i % (NPADM - N)]).reshape(NW, CPT, CH)

    degp = _deg_call(dst_p)
    d0 = degp[:N].reshape(N, 1)
    d1 = degp[NPADD:NPADD + N].reshape(N, 1)

    hw1, hws1 = _dense1(d0, d1, x, W1)
    s1 = _msg_call(hws1, src_p, dst_p)
    hw2, hws2 = _mid(d0, d1, s1, s1, hw1, b1.reshape(1, D), W2)
    s2 = _msg_call(hws2, src_p, dst_p)
    hw3, hws3 = _mid(d0, d1, s2, s2, hw2, b2.reshape(1, D), W3)
    s3 = _msg_call(hws3, src_p, dst_p)

    batch2d = batch.reshape(NB, 1, BK).astype(jnp.int32)
    demo = jnp.zeros((GP, 8), jnp.float32).at[:G].set(demographic)
    return _head(d0, d1, s3, s3, hw3, b3.reshape(1, D), batch2d,
                 demo, fw1[:D], fw1[D:], fb1.reshape(1, 64),
                 fw2, fb2.reshape(1, 32), fw3, fb3.reshape(1, 2))


# speedup vs baseline: 28.2009x; 1.0111x over previous
import functools

import jax
import jax.numpy as jnp
from jax import lax
from jax.experimental import pallas as pl
from jax.experimental.pallas import tpu as pltpu
from jax.experimental.pallas import tpu_sc as plsc

N = 10000
E = 320000
D = 128
G = 100
GP = 104
NC = 2
NS = 16
NW = NC * NS
CH = 128
CPT = 80
EPT = CPT * CH
EPAD = EPT * NW
ZRM = 632
NPADM = NS * ZRM
RLASTM = N - 15 * ZRM
ZRD = 640
NPADD = NS * ZRD
BK = 1000
NB = N // BK

_mesh = plsc.VectorSubcoreMesh(core_axis_name="c", subcore_axis_name="s")


@functools.partial(
    pl.kernel,
    mesh=_mesh,
    out_type=jax.ShapeDtypeStruct((NC * NPADD,), jnp.float32),
    scratch_types=[
        pltpu.VMEM((CPT, CH), jnp.int32),
        pltpu.VMEM((CH,), jnp.float32),
        pltpu.VMEM((ZRD,), jnp.float32),
        pltpu.VMEM_SHARED((NPADD,), jnp.float32),
    ],
)
def _deg_call(dst_hbm, out_hbm, dst_v, ones_v, zbuf, hist):
    c = lax.axis_index("c")
    s = lax.axis_index("s")
    wid = c * NS + s

    for i in range(ZRD // 16):
        zbuf[pl.ds(i * 16, 16)] = jnp.zeros((16,), jnp.float32)
    for i in range(CH // 16):
        ones_v[pl.ds(i * 16, 16)] = jnp.ones((16,), jnp.float32)

    pltpu.sync_copy(dst_hbm.at[wid], dst_v)
    pltpu.sync_copy(zbuf, hist.at[pl.ds(s * ZRD, ZRD)])
    plsc.subcore_barrier()

    def body(j, _):
        pltpu.sync_copy(ones_v, hist.at[dst_v.at[j]], add=True)
        return _

    lax.fori_loop(0, CPT, body, None)
    plsc.subcore_barrier()

    pltpu.sync_copy(hist.at[pl.ds(s * ZRD, ZRD)], zbuf)
    pltpu.sync_copy(zbuf, out_hbm.at[pl.ds(c * NPADD + s * ZRD, ZRD)])


@functools.partial(
    pl.kernel,
    mesh=_mesh,
    out_type=jax.ShapeDtypeStruct((NC * N, D), jnp.float32),
    scratch_types=[
        pltpu.VMEM((CPT, CH), jnp.int32),
        pltpu.VMEM((CH,), jnp.int32),
        pltpu.VMEM((CH,), jnp.int32),
        pltpu.VMEM((CH, D), jnp.float32),
        pltpu.VMEM((CH, D), jnp.float32),
        pltpu.VMEM((8, D), jnp.float32),
        pltpu.VMEM_SHARED((NPADM, D), jnp.float32),
        pltpu.SemaphoreType.DMA,
        pltpu.SemaphoreType.DMA,
        pltpu.SemaphoreType.DMA,
        pltpu.SemaphoreType.DMA,
    ],
)
def _msg_call(hws_hbm, src_hbm, dst_hbm, out_hbm,
              src_v, didx0, didx1, rows0, rows1, zbuf, acc,
              sem0, sem1, dsem0, dsem1):
    c = lax.axis_index("c")
    s = lax.axis_index("s")
    wid = c * NS + s

    for i in range(8):
        for j in range(D // 16):
            zbuf[i, pl.ds(j * 16, 16)] = jnp.zeros((16,), jnp.float32)

    pltpu.sync_copy(src_hbm.at[wid], src_v)

    def zero_body(k, _):
        pltpu.sync_copy(zbuf, acc.at[pl.ds(s * ZRM + k * 8, 8)])
        return _

    lax.fori_loop(0, ZRM // 8, zero_body, None)
    plsc.subcore_barrier()

    pltpu.async_copy(dst_hbm.at[wid, 0], didx0, dsem0)
    pltpu.async_copy(hws_hbm.at[src_v.at[0]], rows0, sem0)

    def body(k, _):
        j0 = 2 * k
        pltpu.async_copy(dst_hbm.at[wid, j0 + 1], didx1, dsem1)
        pltpu.async_copy(hws_hbm.at[src_v.at[j0 + 1]], rows1, sem1)
        pltpu.make_async_copy(dst_hbm.at[wid, j0], didx0, dsem0).wait()
        pltpu.make_async_copy(hws_hbm.at[src_v.at[j0]], rows0, sem0).wait()
        pltpu.sync_copy(rows0, acc.at[didx0], add=True)

        @pl.when(k < CPT // 2 - 1)
        def _():
            pltpu.async_copy(dst_hbm.at[wid, j0 + 2], didx0, dsem0)
            pltpu.async_copy(hws_hbm.at[src_v.at[j0 + 2]], rows0, sem0)

        pltpu.make_async_copy(dst_hbm.at[wid, j0 + 1], didx1, dsem1).wait()
        pltpu.make_async_copy(hws_hbm.at[src_v.at[j0 + 1]], rows1, sem1).wait()
        pltpu.sync_copy(rows1, acc.at[didx1], add=True)
        return _

    lax.fori_loop(0, CPT // 2, body, None)
    plsc.subcore_barrier()

    @pl.when(s < NS - 1)
    def _wb():
        pltpu.sync_copy(acc.at[pl.ds(s * ZRM, ZRM)],
                        out_hbm.at[pl.ds(c * N + s * ZRM, ZRM)])

    @pl.when(s == NS - 1)
    def _wb_last():
        pltpu.sync_copy(acc.at[pl.ds(s * ZRM, RLASTM)],
                        out_hbm.at[pl.ds(c * N + s * ZRM, RLASTM)])


def _dense1_body(d0_ref, d1_ref, x_ref, w_ref, hws_ref):
    deg = d0_ref[...] + d1_ref[...] + 1.0
    dis = lax.rsqrt(deg)
    hw = jnp.dot(x_ref[...], w_ref[...], preferred_element_type=jnp.float32)
    hws_ref[...] = hw * dis


def _dense1(d0, d1, x, w):
    return pl.pallas_call(
        _dense1_body,
        grid=(NB,),
        in_specs=[
            pl.BlockSpec((BK, 1), lambda i: (i, 0)),
            pl.BlockSpec((BK, 1), lambda i: (i, 0)),
            pl.BlockSpec((BK, D), lambda i: (i, 0)),
            pl.BlockSpec((D, D), lambda i: (0, 0)),
        ],
        out_specs=pl.BlockSpec((BK, D), lambda i: (i, 0)),
        out_shape=jax.ShapeDtypeStruct((N, D), jnp.float32),
    )(d0, d1, x, w)


def _mid_body(d0_ref, d1_ref, p0_ref, p1_ref, hwsp_ref, b_ref, w_ref,
              hws_ref):
    deg = d0_ref[...] + d1_ref[...] + 1.0
    dis = lax.rsqrt(deg)
    agg = dis * (p0_ref[...] + p1_ref[...] + hwsp_ref[...])
    h = jnp.maximum(agg + b_ref[...], 0.0)
    hw = jnp.dot(h, w_ref[...], preferred_element_type=jnp.float32)
    hws_ref[...] = hw * dis


def _mid(d0, d1, p0, p1, hwp, b, w):
    return pl.pallas_call(
        _mid_body,
        grid=(NB,),
        in_specs=[
            pl.BlockSpec((BK, 1), lambda i: (i, 0)),
            pl.BlockSpec((BK, 1), lambda i: (i, 0)),
            pl.BlockSpec((BK, D), lambda i: (i, 0)),
            pl.BlockSpec((BK, D), lambda i: (i + NB, 0)),
            pl.BlockSpec((BK, D), lambda i: (i, 0)),
            pl.BlockSpec((1, D), lambda i: (0, 0)),
            pl.BlockSpec((D, D), lambda i: (0, 0)),
        ],
        out_specs=pl.BlockSpec((BK, D), lambda i: (i, 0)),
        out_shape=jax.ShapeDtypeStruct((N, D), jnp.float32),
    )(d0, d1, p0, p1, hwp, b, w)


def _head_body(d0_ref, d1_ref, p0_ref, p1_ref, hwp_ref, b_ref, batch_ref,
               demo_ref, fw1a_ref, fw1b_ref, fb1_ref, fw2_ref, fb2_ref,
               fw3_ref, fb3_ref, out_ref, sum_acc, cnt_acc):
    i = pl.program_id(0)

    @pl.when(i == 0)
    def _init():
        sum_acc[...] = jnp.zeros_like(sum_acc)
        cnt_acc[...] = jnp.zeros_like(cnt_acc)

    deg = d0_ref[...] + d1_ref[...] + 1.0
    dis = lax.rsqrt(deg)
    agg = dis * (p0_ref[...] + p1_ref[...] + hwp_ref[...])
    h = jnp.maximum(agg + b_ref[...], 0.0)

    gid = lax.broadcasted_iota(jnp.int32, (GP, 1), 0)
    mask = (gid == batch_ref[...].reshape(1, BK)).astype(jnp.float32)
    sum_acc[...] += jnp.dot(mask, h, preferred_element_type=jnp.float32)
    cnt_acc[...] += jnp.sum(mask, axis=1, keepdims=True)

    @pl.when(i == NB - 1)
    def _fin():
        pooled = sum_acc[...] / jnp.maximum(cnt_acc[...], 1.0)
        z = jnp.dot(pooled, fw1a_ref[...], preferred_element_type=jnp.float32)
        z += jnp.dot(demo_ref[...], fw1b_ref[...],
                     preferred_element_type=jnp.float32)
        z = jnp.maximum(z + fb1_ref[...], 0.0)
        z = jnp.maximum(
            jnp.dot(z, fw2_ref[...], preferred_element_type=jnp.float32)
            + fb2_ref[...], 0.0)
        z = jnp.dot(z, fw3_ref[...],
                    preferred_element_type=jnp.float32) + fb3_ref[...]
        out_ref[...] = z[:G]


def _head(d0, d1, p0, p1, hwp, b, batch2d, demo,
          fw1a, fw1b, fb1, fw2, fb2, fw3, fb3):
    return pl.pallas_call(
        _head_body,
        grid=(NB,),
        in_specs=[
            pl.BlockSpec((BK, 1), lambda i: (i, 0)),
            pl.BlockSpec((BK, 1), lambda i: (i, 0)),
            pl.BlockSpec((BK, D), lambda i: (i, 0)),
            pl.BlockSpec((BK, D), lambda i: (i + NB, 0)),
            pl.BlockSpec((BK, D), lambda i: (i, 0)),
            pl.BlockSpec((1, D), lambda i: (0, 0)),
            pl.BlockSpec((1, 1, BK), lambda i: (i, 0, 0)),
            pl.BlockSpec((GP, 8), lambda i: (0, 0)),
            pl.BlockSpec((D, 64), lambda i: (0, 0)),
            pl.BlockSpec((8, 64), lambda i: (0, 0)),
            pl.BlockSpec((1, 64), lambda i: (0, 0)),
            pl.BlockSpec((64, 32), lambda i: (0, 0)),
            pl.BlockSpec((1, 32), lambda i: (0, 0)),
            pl.BlockSpec((32, 2), lambda i: (0, 0)),
            pl.BlockSpec((1, 2), lambda i: (0, 0)),
        ],
        out_specs=pl.BlockSpec((G, 2), lambda i: (0, 0)),
        out_shape=jax.ShapeDtypeStruct((G, 2), jnp.float32),
        scratch_shapes=[
            pltpu.VMEM((GP, D), jnp.float32),
            pltpu.VMEM((GP, 1), jnp.float32),
        ],
    )(d0, d1, p0, p1, hwp, b, batch2d, demo,
      fw1a, fw1b, fb1, fw2, fb2, fw3, fb3)


def kernel(x, edge_index, batch, demographic,
           W1, b1, W2, b2, W3, b3, fw1, fb1, fw2, fb2, fw3, fb3):
    src = edge_index[0].astype(jnp.int32)
    dst = edge_index[1].astype(jnp.int32)
    pad = EPAD - E
    pad_i = jnp.arange(pad, dtype=jnp.int32)
    src_p = jnp.concatenate(
        [src, pad_i % N]).reshape(NW, CPT, CH)
    dst_p = jnp.concatenate(
        [dst, N + pad_i % (NPADM - N)]).reshape(NW, CPT, CH)

    degp = _deg_call(dst_p)
    d0 = degp[:N].reshape(N, 1)
    d1 = degp[NPADD:NPADD + N].reshape(N, 1)

    hws1 = _dense1(d0, d1, x, W1)
    s1 = _msg_call(hws1, src_p, dst_p)
    hws2 = _mid(d0, d1, s1, s1, hws1, b1.reshape(1, D), W2)
    s2 = _msg_call(hws2, src_p, dst_p)
    hws3 = _mid(d0, d1, s2, s2, hws2, b2.reshape(1, D), W3)
    s3 = _msg_call(hws3, src_p, dst_p)

    batch2d = batch.reshape(NB, 1, BK).astype(jnp.int32)
    demo = jnp.zeros((GP, 8), jnp.float32).at[:G].set(demographic)
    return _head(d0, d1, s3, s3, hws3, b3.reshape(1, D), batch2d,
                 demo, fw1[:D], fw1[D:], fb1.reshape(1, 64),
                 fw2, fb2.reshape(1, 32), fw3, fb3.reshape(1, 2))


# speedup vs baseline: 28.3583x; 1.0056x over previous
import functools

import jax
import jax.numpy as jnp
from jax import lax
from jax.experimental import pallas as pl
from jax.experimental.pallas import tpu as pltpu
from jax.experimental.pallas import tpu_sc as plsc

N = 10000
E = 320000
D = 128
G = 100
GP = 104
NC = 2
NS = 16
NW = NC * NS
CH = 128
CPT = 80
EPT = CPT * CH
EPAD = EPT * NW
ZRM = 632
NPADM = NS * ZRM
RLASTM = N - 15 * ZRM
ZRD = 640
NPADD = NS * ZRD
BK = 1000
NB = N // BK

_mesh = plsc.VectorSubcoreMesh(core_axis_name="c", subcore_axis_name="s")


@functools.partial(
    pl.kernel,
    mesh=_mesh,
    out_type=jax.ShapeDtypeStruct((NC * NPADD,), jnp.float32),
    scratch_types=[
        pltpu.VMEM((CPT, CH), jnp.int32),
        pltpu.VMEM((CH,), jnp.float32),
        pltpu.VMEM((ZRD,), jnp.float32),
        pltpu.VMEM_SHARED((NPADD,), jnp.float32),
    ],
)
def _deg_call(dst_hbm, out_hbm, dst_v, ones_v, zbuf, hist):
    c = lax.axis_index("c")
    s = lax.axis_index("s")
    wid = c * NS + s

    for i in range(ZRD // 16):
        zbuf[pl.ds(i * 16, 16)] = jnp.zeros((16,), jnp.float32)
    for i in range(CH // 16):
        ones_v[pl.ds(i * 16, 16)] = jnp.ones((16,), jnp.float32)

    pltpu.sync_copy(dst_hbm.at[wid], dst_v)
    pltpu.sync_copy(zbuf, hist.at[pl.ds(s * ZRD, ZRD)])
    plsc.subcore_barrier()

    def body(j, _):
        pltpu.sync_copy(ones_v, hist.at[dst_v.at[j]], add=True)
        return _

    lax.fori_loop(0, CPT, body, None)
    plsc.subcore_barrier()

    pltpu.sync_copy(hist.at[pl.ds(s * ZRD, ZRD)], zbuf)
    pltpu.sync_copy(zbuf, out_hbm.at[pl.ds(c * NPADD + s * ZRD, ZRD)])


@functools.partial(
    pl.kernel,
    mesh=_mesh,
    out_type=jax.ShapeDtypeStruct((NC * N, D), jnp.float32),
    scratch_types=[
        pltpu.VMEM((CPT, CH), jnp.int32),
        pltpu.VMEM((CH,), jnp.int32),
        pltpu.VMEM((CH,), jnp.int32),
        pltpu.VMEM((CH, D), jnp.float32),
        pltpu.VMEM((CH, D), jnp.float32),
        pltpu.VMEM((8, D), jnp.float32),
        pltpu.VMEM_SHARED((NPADM, D), jnp.float32),
        pltpu.SemaphoreType.DMA,
        pltpu.SemaphoreType.DMA,
        pltpu.SemaphoreType.DMA,
        pltpu.SemaphoreType.DMA,
    ],
)
def _msg_call(hws_hbm, src_hbm, dst_hbm, out_hbm,
              src_v, didx0, didx1, rows0, rows1, zbuf, acc,
              sem0, sem1, dsem0, dsem1):
    c = lax.axis_index("c")
    s = lax.axis_index("s")
    wid = c * NS + s

    for i in range(8):
        for j in range(D // 16):
            zbuf[i, pl.ds(j * 16, 16)] = jnp.zeros((16,), jnp.float32)

    pltpu.sync_copy(src_hbm.at[wid], src_v)

    def zero_body(k, _):
        pltpu.sync_copy(zbuf, acc.at[pl.ds(s * ZRM + k * 8, 8)])
        return _

    lax.fori_loop(0, ZRM // 8, zero_body, None)
    plsc.subcore_barrier()

    pltpu.async_copy(dst_hbm.at[wid, 0], didx0, dsem0)
    pltpu.async_copy(hws_hbm.at[src_v.at[0]], rows0, sem0)

    def body(k, _):
        j0 = 2 * k
        pltpu.async_copy(dst_hbm.at[wid, j0 + 1], didx1, dsem1)
        pltpu.async_copy(hws_hbm.at[src_v.at[j0 + 1]], rows1, sem1)
        pltpu.make_async_copy(dst_hbm.at[wid, j0], didx0, dsem0).wait()
        pltpu.make_async_copy(hws_hbm.at[src_v.at[j0]], rows0, sem0).wait()
        pltpu.sync_copy(rows0, acc.at[didx0], add=True)

        @pl.when(k < CPT // 2 - 1)
        def _():
            pltpu.async_copy(dst_hbm.at[wid, j0 + 2], didx0, dsem0)
            pltpu.async_copy(hws_hbm.at[src_v.at[j0 + 2]], rows0, sem0)

        pltpu.make_async_copy(dst_hbm.at[wid, j0 + 1], didx1, dsem1).wait()
        pltpu.make_async_copy(hws_hbm.at[src_v.at[j0 + 1]], rows1, sem1).wait()
        pltpu.sync_copy(rows1, acc.at[didx1], add=True)
        return _

    lax.fori_loop(0, CPT // 2, body, None)
    plsc.subcore_barrier()

    @pl.when(s < NS - 1)
    def _wb():
        pltpu.sync_copy(acc.at[pl.ds(s * ZRM, ZRM)],
                        out_hbm.at[pl.ds(c * N + s * ZRM, ZRM)])

    @pl.when(s == NS - 1)
    def _wb_last():
        pltpu.sync_copy(acc.at[pl.ds(s * ZRM, RLASTM)],
                        out_hbm.at[pl.ds(c * N + s * ZRM, RLASTM)])


def _matmul1_body(x_ref, w_ref, hw_ref):
    hw_ref[...] = jnp.dot(x_ref[...], w_ref[...],
                          preferred_element_type=jnp.float32)


def _matmul1(x, w):
    return pl.pallas_call(
        _matmul1_body,
        grid=(NB,),
        in_specs=[
            pl.BlockSpec((BK, D), lambda i: (i, 0)),
            pl.BlockSpec((D, D), lambda i: (0, 0)),
        ],
        out_specs=pl.BlockSpec((BK, D), lambda i: (i, 0)),
        out_shape=jax.ShapeDtypeStruct((N, D), jnp.float32),
    )(x, w)


def _scale_body(d0_ref, d1_ref, hw_ref, hws_ref):
    deg = d0_ref[...] + d1_ref[...] + 1.0
    hws_ref[...] = hw_ref[...] * lax.rsqrt(deg)


def _scale(d0, d1, hw):
    return pl.pallas_call(
        _scale_body,
        grid=(NB,),
        in_specs=[
            pl.BlockSpec((BK, 1), lambda i: (i, 0)),
            pl.BlockSpec((BK, 1), lambda i: (i, 0)),
            pl.BlockSpec((BK, D), lambda i: (i, 0)),
        ],
        out_specs=pl.BlockSpec((BK, D), lambda i: (i, 0)),
        out_shape=jax.ShapeDtypeStruct((N, D), jnp.float32),
    )(d0, d1, hw)


def _mid_body(d0_ref, d1_ref, p0_ref, p1_ref, hwsp_ref, b_ref, w_ref,
              hws_ref):
    deg = d0_ref[...] + d1_ref[...] + 1.0
    dis = lax.rsqrt(deg)
    agg = dis * (p0_ref[...] + p1_ref[...] + hwsp_ref[...])
    h = jnp.maximum(agg + b_ref[...], 0.0)
    hw = jnp.dot(h, w_ref[...], preferred_element_type=jnp.float32)
    hws_ref[...] = hw * dis


def _mid(d0, d1, p0, p1, hwp, b, w):
    return pl.pallas_call(
        _mid_body,
        grid=(NB,),
        in_specs=[
            pl.BlockSpec((BK, 1), lambda i: (i, 0)),
            pl.BlockSpec((BK, 1), lambda i: (i, 0)),
            pl.BlockSpec((BK, D), lambda i: (i, 0)),
            pl.BlockSpec((BK, D), lambda i: (i + NB, 0)),
            pl.BlockSpec((BK, D), lambda i: (i, 0)),
            pl.BlockSpec((1, D), lambda i: (0, 0)),
            pl.BlockSpec((D, D), lambda i: (0, 0)),
        ],
        out_specs=pl.BlockSpec((BK, D), lambda i: (i, 0)),
        out_shape=jax.ShapeDtypeStruct((N, D), jnp.float32),
    )(d0, d1, p0, p1, hwp, b, w)


def _head_body(d0_ref, d1_ref, p0_ref, p1_ref, hwp_ref, b_ref, batch_ref,
               demo_ref, fw1a_ref, fw1b_ref, fb1_ref, fw2_ref, fb2_ref,
               fw3_ref, fb3_ref, out_ref, sum_acc, cnt_acc):
    i = pl.program_id(0)

    @pl.when(i == 0)
    def _init():
        sum_acc[...] = jnp.zeros_like(sum_acc)
        cnt_acc[...] = jnp.zeros_like(cnt_acc)

    deg = d0_ref[...] + d1_ref[...] + 1.0
    dis = lax.rsqrt(deg)
    agg = dis * (p0_ref[...] + p1_ref[...] + hwp_ref[...])
    h = jnp.maximum(agg + b_ref[...], 0.0)

    gid = lax.broadcasted_iota(jnp.int32, (GP, 1), 0)
    mask = (gid == batch_ref[...].reshape(1, BK)).astype(jnp.float32)
    sum_acc[...] += jnp.dot(mask, h, preferred_element_type=jnp.float32)
    cnt_acc[...] += jnp.sum(mask, axis=1, keepdims=True)

    @pl.when(i == NB - 1)
    def _fin():
        pooled = sum_acc[...] / jnp.maximum(cnt_acc[...], 1.0)
        z = jnp.dot(pooled, fw1a_ref[...], preferred_element_type=jnp.float32)
        z += jnp.dot(demo_ref[...], fw1b_ref[...],
                     preferred_element_type=jnp.float32)
        z = jnp.maximum(z + fb1_ref[...], 0.0)
        z = jnp.maximum(
            jnp.dot(z, fw2_ref[...], preferred_element_type=jnp.float32)
            + fb2_ref[...], 0.0)
        z = jnp.dot(z, fw3_ref[...],
                    preferred_element_type=jnp.float32) + fb3_ref[...]
        out_ref[...] = z[:G]


def _head(d0, d1, p0, p1, hwp, b, batch2d, demo,
          fw1a, fw1b, fb1, fw2, fb2, fw3, fb3):
    return pl.pallas_call(
        _head_body,
        grid=(NB,),
        in_specs=[
            pl.BlockSpec((BK, 1), lambda i: (i, 0)),
            pl.BlockSpec((BK, 1), lambda i: (i, 0)),
            pl.BlockSpec((BK, D), lambda i: (i, 0)),
            pl.BlockSpec((BK, D), lambda i: (i + NB, 0)),
            pl.BlockSpec((BK, D), lambda i: (i, 0)),
            pl.BlockSpec((1, D), lambda i: (0, 0)),
            pl.BlockSpec((1, 1, BK), lambda i: (i, 0, 0)),
            pl.BlockSpec((GP, 8), lambda i: (0, 0)),
            pl.BlockSpec((D, 64), lambda i: (0, 0)),
            pl.BlockSpec((8, 64), lambda i: (0, 0)),
            pl.BlockSpec((1, 64), lambda i: (0, 0)),
            pl.BlockSpec((64, 32), lambda i: (0, 0)),
            pl.BlockSpec((1, 32), lambda i: (0, 0)),
            pl.BlockSpec((32, 2), lambda i: (0, 0)),
            pl.BlockSpec((1, 2), lambda i: (0, 0)),
        ],
        out_specs=pl.BlockSpec((G, 2), lambda i: (0, 0)),
        out_shape=jax.ShapeDtypeStruct((G, 2), jnp.float32),
        scratch_shapes=[
            pltpu.VMEM((GP, D), jnp.float32),
            pltpu.VMEM((GP, 1), jnp.float32),
        ],
    )(d0, d1, p0, p1, hwp, b, batch2d, demo,
      fw1a, fw1b, fb1, fw2, fb2, fw3, fb3)


def kernel(x, edge_index, batch, demographic,
           W1, b1, W2, b2, W3, b3, fw1, fb1, fw2, fb2, fw3, fb3):
    src = edge_index[0].astype(jnp.int32)
    dst = edge_index[1].astype(jnp.int32)
    pad = EPAD - E
    pad_i = jnp.arange(pad, dtype=jnp.int32)
    src_p = jnp.concatenate(
        [src, pad_i % N]).reshape(NW, CPT, CH)
    dst_p = jnp.concatenate(
        [dst, N + pad_i % (NPADM - N)]).reshape(NW, CPT, CH)

    degp = _deg_call(dst_p)
    d0 = degp[:N].reshape(N, 1)
    d1 = degp[NPADD:NPADD + N].reshape(N, 1)

    hws1 = _scale(d0, d1, _matmul1(x, W1))
    s1 = _msg_call(hws1, src_p, dst_p)
    hws2 = _mid(d0, d1, s1, s1, hws1, b1.reshape(1, D), W2)
    s2 = _msg_call(hws2, src_p, dst_p)
    hws3 = _mid(d0, d1, s2, s2, hws2, b2.reshape(1, D), W3)
    s3 = _msg_call(hws3, src_p, dst_p)

    batch2d = batch.reshape(NB, 1, BK).astype(jnp.int32)
    demo = jnp.zeros((GP, 8), jnp.float32).at[:G].set(demographic)
    return _head(d0, d1, s3, s3, hws3, b3.reshape(1, D), batch2d,
                 demo, fw1[:D], fw1[D:], fb1.reshape(1, 64),
                 fw2, fb2.reshape(1, 32), fw3, fb3.reshape(1, 2))
